# baseline (speedup 1.0000x reference)
import functools

import jax
import jax.numpy as jnp
from jax import lax
from jax.experimental import pallas as pl
from jax.experimental.pallas import tpu as pltpu
from jax.experimental.pallas import tpu_sc as plsc

N = 10000
E = 320000
FIN = 128
DH = 256
C = 40
CP = 128
NP = 10240
NW = 32
EP = 344064
EPW = EP // NW
CHUNK = 128
NCH = EPW // CHUNK
RPT = NP // 16
NB = 2
NI = 3
NGRP = NCH // 6
DGRP = NCH // 2


def _mesh():
    return plsc.VectorSubcoreMesh(core_axis_name="c", subcore_axis_name="s")


def _prop_body(y_hbm, src_hbm, dst_hbm, out_hbm, wid, sid, cid,
               rows, si_v, di_v, acc_sh, gsem, ssem, isem):
    def _idx(i, b):
        pltpu.async_copy(src_hbm.at[wid, i], si_v.at[b], isem[b])
        pltpu.async_copy(dst_hbm.at[wid, i], di_v.at[b], isem[b])

    def _idx_wait(i, b):
        pltpu.make_async_copy(src_hbm.at[wid, i], si_v.at[b], isem[b]).wait()
        pltpu.make_async_copy(dst_hbm.at[wid, i], di_v.at[b], isem[b]).wait()

    def _gather(ib, rb):
        pltpu.async_copy(y_hbm.at[si_v.at[ib]], rows[rb], gsem[rb])

    def _gather_wait(ib, rb):
        pltpu.make_async_copy(y_hbm.at[si_v.at[ib]], rows[rb],
                              gsem[rb]).wait()

    def _scat(ib, rb):
        pltpu.async_copy(rows[rb], acc_sh.at[di_v.at[ib]], ssem[rb],
                         add=True)

    def _scat_wait(ib, rb):
        pltpu.make_async_copy(rows[rb], acc_sh.at[di_v.at[ib]],
                              ssem[rb]).wait()

    _idx(0, 0)
    _idx(1, 1)
    _idx_wait(0, 0)
    _gather(0, 0)

    def _grp(k, _):
        for j in range(6):
            i = k * 6 + j
            rb, ib = j % 2, j % 3
            rb1, ib1 = (j + 1) % 2, (j + 1) % 3
            ib2 = (j + 2) % 3
            _gather_wait(ib, rb)
            _scat(ib, rb)
            if j == 0:
                @pl.when(k > 0)
                def _():
                    _scat_wait(ib1, rb1)

                _idx(i + 2, ib2)
            else:
                _scat_wait(ib1, rb1)
                if j < 4:
                    _idx(i + 2, ib2)
                else:
                    @pl.when(k < NGRP - 1)
                    def _():
                        _idx(i + 2, ib2)
            if j < 5:
                _idx_wait(i + 1, ib1)
                _gather(ib1, rb1)
            else:
                @pl.when(k < NGRP - 1)
                def _():
                    _idx_wait(i + 1, ib1)
                    _gather(ib1, rb1)
        return 0

    lax.fori_loop(0, NGRP, _grp, 0)
    _scat_wait((NCH - 1) % 3, (NCH - 1) % 2)
    plsc.subcore_barrier()
    pltpu.sync_copy(acc_sh.at[pl.ds(sid * RPT, RPT)],
                    out_hbm.at[cid, pl.ds(sid * RPT, RPT)])


@functools.partial(
    pl.kernel,
    out_type=(jax.ShapeDtypeStruct((NP,), jnp.float32),
              jax.ShapeDtypeStruct((NP, FIN), jnp.float32),
              jax.ShapeDtypeStruct((2, NP, FIN), jnp.float32)),
    mesh=_mesh(),
    scratch_types=[
        pltpu.VMEM((NCH, CHUNK), jnp.int32),
        pltpu.VMEM((CHUNK,), jnp.float32),
        pltpu.VMEM((RPT,), jnp.float32),
        pltpu.VMEM((NI, CHUNK), jnp.int32),
        pltpu.VMEM((NI, CHUNK), jnp.int32),
        pltpu.VMEM((CHUNK, FIN), jnp.float32),
        pltpu.VMEM((CHUNK, FIN), jnp.float32),
        pltpu.VMEM_SHARED((NP,), jnp.float32),
        pltpu.VMEM_SHARED((NP, FIN), jnp.float32),
        pltpu.SemaphoreType.DMA,
        pltpu.SemaphoreType.DMA,
        pltpu.SemaphoreType.DMA,
        pltpu.SemaphoreType.DMA,
        pltpu.SemaphoreType.DMA,
        pltpu.SemaphoreType.DMA,
        pltpu.SemaphoreType.DMA,
    ],
)
def _fused_sc(x_hbm, src_hbm, dst_hbm, zero_hbm,
              dinv_hbm, y_hbm, out_hbm,
              slab_v, ones_v, dbuf_v, si_v, di_v, r0, r1,
              deg_sh, acc_sh, g0, g1, s0, s1, i0, i1, i2):
    cid = lax.axis_index("c")
    sid = lax.axis_index("s")
    wid = cid * 16 + sid
    rows = (r0, r1)
    gsem = (g0, g1)
    ssem = (s0, s1)
    isem = (i0, i1, i2)

    def _fill(i, _):
        ones_v[pl.ds(i * 16, 16)] = jnp.ones((16,), jnp.float32)
        dbuf_v[pl.ds(i * 16, 16)] = jnp.zeros((16,), jnp.float32)
        return 0

    lax.fori_loop(0, CHUNK // 16, _fill, 0)

    def _zero(i, _):
        dbuf_v[pl.ds(i * 16, 16)] = jnp.zeros((16,), jnp.float32)
        return 0

    lax.fori_loop(0, RPT // 16, _zero, 0)
    pltpu.sync_copy(dbuf_v, deg_sh.at[pl.ds(sid * RPT, RPT)])
    pltpu.sync_copy(zero_hbm.at[pl.ds(sid * RPT, RPT)],
                    acc_sh.at[pl.ds(sid * RPT, RPT)])
    plsc.subcore_barrier()

    for w2 in range(2):
        pltpu.sync_copy(dst_hbm.at[sid * 2 + w2], slab_v)

        def _dgrp(k, _):
            for b in range(2):
                i = k * 2 + b

                @pl.when(k > 0)
                def _():
                    pltpu.make_async_copy(
                        ones_v, deg_sh.at[slab_v.at[i]], ssem[b]).wait()

                pltpu.async_copy(ones_v, deg_sh.at[slab_v.at[i]], ssem[b],
                                 add=True)
            return 0

        lax.fori_loop(0, DGRP, _dgrp, 0)
        for b in range(2):
            pltpu.make_async_copy(ones_v, deg_sh.at[slab_v.at[0]],
                                  ssem[b]).wait()
    plsc.subcore_barrier()

    pltpu.sync_copy(deg_sh.at[pl.ds(sid * RPT, RPT)], dbuf_v)

    def _rsq(g, _):
        d = dbuf_v[pl.ds(g * 16, 16)]
        yv = jnp.full((16,), 2.0 ** -9.5, jnp.float32)
        for kk in range(9, 0, -1):
            yv = jnp.where(d < jnp.float32(4.0 ** kk),
                           jnp.float32(2.0 ** -(kk - 0.5)), yv)
        for _unused in range(6):
            yv = yv * (1.5 - 0.5 * d * yv * yv)
        dbuf_v[pl.ds(g * 16, 16)] = jnp.where(d > 0.5, yv, 0.0)
        return 0

    lax.fori_loop(0, RPT // 16, _rsq, 0)
    pltpu.sync_copy(dbuf_v, dinv_hbm.at[pl.ds(sid * RPT, RPT)])

    def _sbatch(bt, _):
        base = sid * RPT + bt * CHUNK
        pltpu.sync_copy(x_hbm.at[pl.ds(base, CHUNK)], r0)

        def _sgrp(g, _):
            dgrp = dbuf_v[pl.ds(bt * CHUNK + g * 16, 16)]
            for l in range(16):
                sv = lax.gather(
                    dgrp, jnp.full((16, 1), l, jnp.int32),
                    lax.GatherDimensionNumbers(
                        offset_dims=(), collapsed_slice_dims=(0,),
                        start_index_map=(0,)),
                    (1,), mode=lax.GatherScatterMode.PROMISE_IN_BOUNDS)
                r = g * 16 + l
                for c in range(FIN // 16):
                    r0[r, pl.ds(c * 16, 16)] = r0[r, pl.ds(c * 16, 16)] * sv
            return 0

        lax.fori_loop(0, CHUNK // 16, _sgrp, 0)
        pltpu.sync_copy(r0, y_hbm.at[pl.ds(base, CHUNK)])
        return 0

    lax.fori_loop(0, RPT // CHUNK, _sbatch, 0)
    plsc.subcore_barrier()

    _prop_body(y_hbm, src_hbm, dst_hbm, out_hbm, wid, sid, cid,
               rows, si_v, di_v, acc_sh, gsem, ssem, isem)


def _make_prop(D):
    @functools.partial(
        pl.kernel,
        out_type=jax.ShapeDtypeStruct((2, NP, D), jnp.float32),
        mesh=_mesh(),
        scratch_types=[
            pltpu.VMEM((NI, CHUNK), jnp.int32),
            pltpu.VMEM((NI, CHUNK), jnp.int32),
            pltpu.VMEM((CHUNK, D), jnp.float32),
            pltpu.VMEM((CHUNK, D), jnp.float32),
            pltpu.VMEM_SHARED((NP, D), jnp.float32),
            pltpu.SemaphoreType.DMA,
            pltpu.SemaphoreType.DMA,
            pltpu.SemaphoreType.DMA,
            pltpu.SemaphoreType.DMA,
            pltpu.SemaphoreType.DMA,
            pltpu.SemaphoreType.DMA,
            pltpu.SemaphoreType.DMA,
        ],
    )
    def _prop(y_hbm, src_hbm, dst_hbm, zero_hbm, out_hbm,
              si_v, di_v, r0, r1, acc_sh, g0, g1, s0, s1, i0, i1, i2):
        cid = lax.axis_index("c")
        sid = lax.axis_index("s")
        wid = cid * 16 + sid
        pltpu.sync_copy(zero_hbm.at[pl.ds(sid * RPT, RPT)],
                        acc_sh.at[pl.ds(sid * RPT, RPT)])
        plsc.subcore_barrier()
        _prop_body(y_hbm, src_hbm, dst_hbm, out_hbm, wid, sid, cid,
                   (r0, r1), si_v, di_v, acc_sh, (g0, g1), (s0, s1),
                   (i0, i1, i2))

    return _prop


_prop_fin = _make_prop(FIN)
_prop_cp = _prop_fin


def _mm_body(acc_ref, dinv_ref, w1_ref, b1_ref, w2_ref, y2_ref):
    dinv = dinv_ref[...]
    p1 = dinv * (acc_ref[0] + acc_ref[1])
    h = jnp.maximum(
        lax.dot_general(p1, w1_ref[...], (((1,), (0,)), ((), ())),
                        precision=lax.Precision.HIGHEST,
                        preferred_element_type=jnp.float32) + b1_ref[...],
        0.0)
    g = lax.dot_general(h, w2_ref[...], (((1,), (0,)), ((), ())),
                        precision=lax.Precision.HIGHEST,
                        preferred_element_type=jnp.float32)
    y2_ref[...] = dinv * g


def _mm(acc1, dinv, W1, b1, W2p):
    blk = 1024
    grid = NP // blk
    return pl.pallas_call(
        _mm_body,
        grid=(grid,),
        in_specs=[
            pl.BlockSpec((2, blk, FIN), lambda i: (0, i, 0)),
            pl.BlockSpec((blk, 1), lambda i: (i, 0)),
            pl.BlockSpec((FIN, DH), lambda i: (0, 0)),
            pl.BlockSpec((1, DH), lambda i: (0, 0)),
            pl.BlockSpec((DH, CP), lambda i: (0, 0)),
        ],
        out_specs=pl.BlockSpec((blk, CP), lambda i: (i, 0)),
        out_shape=jax.ShapeDtypeStruct((NP, CP), jnp.float32),
    )(acc1, dinv, W1, b1, W2p)


def _smax_body(acc_ref, dinv_ref, b2_ref, out_ref):
    z = dinv_ref[...] * (acc_ref[0, :, :C] + acc_ref[1, :, :C]) + b2_ref[...]
    m = jnp.max(z, axis=1, keepdims=True)
    e = jnp.exp(z - m)
    s = jnp.sum(e, axis=1, keepdims=True)
    out_ref[...] = z - (m + jnp.log(s))


def _smax(acc2, dinv, b2p):
    blk = 1000
    grid = N // blk
    return pl.pallas_call(
        _smax_body,
        grid=(grid,),
        in_specs=[
            pl.BlockSpec((2, blk, CP), lambda i: (0, i, 0)),
            pl.BlockSpec((blk, 1), lambda i: (i, 0)),
            pl.BlockSpec((1, C), lambda i: (0, 0)),
        ],
        out_specs=pl.BlockSpec((blk, C), lambda i: (i, 0)),
        out_shape=jax.ShapeDtypeStruct((N, C), jnp.float32),
    )(acc2, dinv, b2p)


def kernel(x, edge_index, W1, b1, W2, b2):
    src = edge_index[0].astype(jnp.int32)
    dst = edge_index[1].astype(jnp.int32)
    loops = jnp.arange(N, dtype=jnp.int32)
    padr = jnp.arange(EP - E - N, dtype=jnp.int32)
    src_all = jnp.concatenate([src, loops, padr % N])
    dst_all = jnp.concatenate([dst, loops, N + padr % (NP - N)])
    srcp = src_all.reshape(EPW, NW).T.reshape(NW, NCH, CHUNK)
    dstp = dst_all.reshape(EPW, NW).T.reshape(NW, NCH, CHUNK)
    xp = jnp.pad(x, ((0, NP - N), (0, 0)))
    z_fin = jnp.zeros((NP, FIN), jnp.float32)
    W2p = jnp.pad(W2, ((0, 0), (0, CP - C)))
    b2p = b2.reshape(1, C)

    dinv1d, _y1, acc1 = _fused_sc(xp, srcp, dstp, z_fin)
    dinv = dinv1d.reshape(NP, 1)
    y2 = _mm(acc1, dinv, W1, b1.reshape(1, DH), W2p)
    acc2 = _prop_cp(y2, srcp, dstp, z_fin)
    return _smax(acc2, dinv, b2p)


# speedup vs baseline: 1.0740x; 1.0740x over previous
import functools

import jax
import jax.numpy as jnp
from jax import lax
from jax.experimental import pallas as pl
from jax.experimental.pallas import tpu as pltpu
from jax.experimental.pallas import tpu_sc as plsc

N = 10000
E = 320000
FIN = 128
DH = 256
C = 40
CP = 128
NP = 10240
NW = 32
EP = 344064
EPW = EP // NW
CHUNK = 128
NCH = EPW // CHUNK
RPT = NP // 16
NB = 2
NI = 3
NGRP = NCH // 6
DGRP = NCH // 2


def _mesh():
    return plsc.VectorSubcoreMesh(core_axis_name="c", subcore_axis_name="s")


def _prop_body(y_hbm, src_hbm, dst_hbm, out_hbm, wid, sid, cid,
               rows, si_v, di_v, acc_sh, gsem, ssem, isem):
    def _idx(i, b):
        pltpu.async_copy(src_hbm.at[wid, i], si_v.at[b], isem[b])
        pltpu.async_copy(dst_hbm.at[wid, i], di_v.at[b], isem[b])

    def _idx_wait(i, b):
        pltpu.make_async_copy(src_hbm.at[wid, i], si_v.at[b], isem[b]).wait()
        pltpu.make_async_copy(dst_hbm.at[wid, i], di_v.at[b], isem[b]).wait()

    def _gather(ib, rb):
        pltpu.async_copy(y_hbm.at[si_v.at[ib]], rows[rb], gsem[rb])

    def _gather_wait(ib, rb):
        pltpu.make_async_copy(y_hbm.at[si_v.at[ib]], rows[rb],
                              gsem[rb]).wait()

    def _scat(ib, rb):
        pltpu.async_copy(rows[rb], acc_sh.at[di_v.at[ib]], ssem[rb],
                         add=True)

    def _scat_wait(ib, rb):
        pltpu.make_async_copy(rows[rb], acc_sh.at[di_v.at[ib]],
                              ssem[rb]).wait()

    _idx(0, 0)
    _idx(1, 1)
    _idx_wait(0, 0)
    _gather(0, 0)

    def _grp(k, _):
        for j in range(6):
            i = k * 6 + j
            rb, ib = j % 2, j % 3
            rb1, ib1 = (j + 1) % 2, (j + 1) % 3
            ib2 = (j + 2) % 3
            _gather_wait(ib, rb)
            _scat(ib, rb)
            if j == 0:
                @pl.when(k > 0)
                def _():
                    _scat_wait(ib1, rb1)

                _idx(i + 2, ib2)
            else:
                _scat_wait(ib1, rb1)
                if j < 4:
                    _idx(i + 2, ib2)
                else:
                    @pl.when(k < NGRP - 1)
                    def _():
                        _idx(i + 2, ib2)
            if j < 5:
                _idx_wait(i + 1, ib1)
                _gather(ib1, rb1)
            else:
                @pl.when(k < NGRP - 1)
                def _():
                    _idx_wait(i + 1, ib1)
                    _gather(ib1, rb1)
        return 0

    lax.fori_loop(0, NGRP, _grp, 0)
    _scat_wait((NCH - 1) % 3, (NCH - 1) % 2)
    plsc.subcore_barrier()
    pltpu.sync_copy(acc_sh.at[pl.ds(sid * RPT, RPT)],
                    out_hbm.at[cid, pl.ds(sid * RPT, RPT)])


@functools.partial(
    pl.kernel,
    out_type=jax.ShapeDtypeStruct((2, NP), jnp.float32),
    mesh=_mesh(),
    scratch_types=[
        pltpu.VMEM((NCH, CHUNK), jnp.int32),
        pltpu.VMEM((CHUNK,), jnp.float32),
        pltpu.VMEM((RPT,), jnp.float32),
        pltpu.VMEM_SHARED((NP,), jnp.float32),
        pltpu.SemaphoreType.DMA,
        pltpu.SemaphoreType.DMA,
    ],
)
def _deg_sc(dst_hbm, out_hbm, slab_v, ones_v, z_v, acc_sh, s0, s1):
    cid = lax.axis_index("c")
    sid = lax.axis_index("s")
    wid = cid * 16 + sid
    ssem = (s0, s1)

    def _fill(i, _):
        ones_v[pl.ds(i * 16, 16)] = jnp.ones((16,), jnp.float32)
        return 0

    lax.fori_loop(0, CHUNK // 16, _fill, 0)

    def _zero(i, _):
        z_v[pl.ds(i * 16, 16)] = jnp.zeros((16,), jnp.float32)
        return 0

    lax.fori_loop(0, RPT // 16, _zero, 0)
    pltpu.sync_copy(dst_hbm.at[wid], slab_v)
    pltpu.sync_copy(z_v, acc_sh.at[pl.ds(sid * RPT, RPT)])
    plsc.subcore_barrier()

    def _grp(k, _):
        for b in range(2):
            i = k * 2 + b

            @pl.when(k > 0)
            def _():
                pltpu.make_async_copy(
                    ones_v, acc_sh.at[slab_v.at[i]], ssem[b]).wait()

            pltpu.async_copy(ones_v, acc_sh.at[slab_v.at[i]], ssem[b],
                             add=True)
        return 0

    lax.fori_loop(0, DGRP, _grp, 0)
    for b in range(2):
        pltpu.make_async_copy(ones_v, acc_sh.at[slab_v.at[0]], ssem[b]).wait()
    plsc.subcore_barrier()
    pltpu.sync_copy(acc_sh.at[pl.ds(sid * RPT, RPT)],
                    out_hbm.at[cid, pl.ds(sid * RPT, RPT)])


def _make_prop(D):
    @functools.partial(
        pl.kernel,
        out_type=jax.ShapeDtypeStruct((2, NP, D), jnp.float32),
        mesh=_mesh(),
        scratch_types=[
            pltpu.VMEM((NI, CHUNK), jnp.int32),
            pltpu.VMEM((NI, CHUNK), jnp.int32),
            pltpu.VMEM((CHUNK, D), jnp.float32),
            pltpu.VMEM((CHUNK, D), jnp.float32),
            pltpu.VMEM_SHARED((NP, D), jnp.float32),
            pltpu.SemaphoreType.DMA,
            pltpu.SemaphoreType.DMA,
            pltpu.SemaphoreType.DMA,
            pltpu.SemaphoreType.DMA,
            pltpu.SemaphoreType.DMA,
            pltpu.SemaphoreType.DMA,
            pltpu.SemaphoreType.DMA,
        ],
    )
    def _prop(y_hbm, src_hbm, dst_hbm, zero_hbm, out_hbm,
              si_v, di_v, r0, r1, acc_sh, g0, g1, s0, s1, i0, i1, i2):
        cid = lax.axis_index("c")
        sid = lax.axis_index("s")
        wid = cid * 16 + sid
        pltpu.sync_copy(zero_hbm.at[pl.ds(sid * RPT, RPT)],
                        acc_sh.at[pl.ds(sid * RPT, RPT)])
        plsc.subcore_barrier()
        _prop_body(y_hbm, src_hbm, dst_hbm, out_hbm, wid, sid, cid,
                   (r0, r1), si_v, di_v, acc_sh, (g0, g1), (s0, s1),
                   (i0, i1, i2))

    return _prop


_prop_fin = _make_prop(FIN)
_prop_cp = _prop_fin


def _scale_body(deg_ref, x_ref, dinv_ref, y_ref):
    deg = deg_ref[0] + deg_ref[1]
    dinv = jnp.where(deg > 0, lax.rsqrt(deg), 0.0)
    dinv_ref[...] = dinv[:, None]
    y_ref[...] = x_ref[...] * dinv[:, None]


def _scale(deg01, xp):
    return pl.pallas_call(
        _scale_body,
        out_shape=(jax.ShapeDtypeStruct((NP, 1), jnp.float32),
                   jax.ShapeDtypeStruct((NP, FIN), jnp.float32)),
    )(deg01, xp)


def _mm_body(acc_ref, dinv_ref, w1_ref, b1_ref, w2_ref, y2_ref):
    dinv = dinv_ref[...]
    p1 = dinv * (acc_ref[0] + acc_ref[1])
    h = jnp.maximum(
        lax.dot_general(p1, w1_ref[...], (((1,), (0,)), ((), ())),
                        precision=lax.Precision.HIGHEST,
                        preferred_element_type=jnp.float32) + b1_ref[...],
        0.0)
    g = lax.dot_general(h, w2_ref[...], (((1,), (0,)), ((), ())),
                        precision=lax.Precision.HIGHEST,
                        preferred_element_type=jnp.float32)
    y2_ref[...] = dinv * g


def _mm(acc1, dinv, W1, b1, W2p):
    blk = 1024
    grid = NP // blk
    return pl.pallas_call(
        _mm_body,
        grid=(grid,),
        in_specs=[
            pl.BlockSpec((2, blk, FIN), lambda i: (0, i, 0)),
            pl.BlockSpec((blk, 1), lambda i: (i, 0)),
            pl.BlockSpec((FIN, DH), lambda i: (0, 0)),
            pl.BlockSpec((1, DH), lambda i: (0, 0)),
            pl.BlockSpec((DH, CP), lambda i: (0, 0)),
        ],
        out_specs=pl.BlockSpec((blk, CP), lambda i: (i, 0)),
        out_shape=jax.ShapeDtypeStruct((NP, CP), jnp.float32),
    )(acc1, dinv, W1, b1, W2p)


def _smax_body(acc_ref, dinv_ref, b2_ref, out_ref):
    z = dinv_ref[...] * (acc_ref[0, :, :C] + acc_ref[1, :, :C]) + b2_ref[...]
    m = jnp.max(z, axis=1, keepdims=True)
    e = jnp.exp(z - m)
    s = jnp.sum(e, axis=1, keepdims=True)
    out_ref[...] = z - (m + jnp.log(s))


def _smax(acc2, dinv, b2p):
    blk = 1000
    grid = N // blk
    return pl.pallas_call(
        _smax_body,
        grid=(grid,),
        in_specs=[
            pl.BlockSpec((2, blk, CP), lambda i: (0, i, 0)),
            pl.BlockSpec((blk, 1), lambda i: (i, 0)),
            pl.BlockSpec((1, C), lambda i: (0, 0)),
        ],
        out_specs=pl.BlockSpec((blk, C), lambda i: (i, 0)),
        out_shape=jax.ShapeDtypeStruct((N, C), jnp.float32),
    )(acc2, dinv, b2p)


def kernel(x, edge_index, W1, b1, W2, b2):
    src = edge_index[0].astype(jnp.int32)
    dst = edge_index[1].astype(jnp.int32)
    loops = jnp.arange(N, dtype=jnp.int32)
    padr = jnp.arange(EP - E - N, dtype=jnp.int32)
    src_all = jnp.concatenate([src, loops, padr % N])
    dst_all = jnp.concatenate([dst, loops, N + padr % (NP - N)])
    srcp = src_all.reshape(EPW, NW).T.reshape(NW, NCH, CHUNK)
    dstp = dst_all.reshape(EPW, NW).T.reshape(NW, NCH, CHUNK)
    xp = jnp.pad(x, ((0, NP - N), (0, 0)))
    z_fin = jnp.zeros((NP, FIN), jnp.float32)
    W2p = jnp.pad(W2, ((0, 0), (0, CP - C)))
    b2p = b2.reshape(1, C)

    deg01 = _deg_sc(dstp)
    dinv, y1 = _scale(deg01, xp)
    acc1 = _prop_fin(y1, srcp, dstp, z_fin)
    y2 = _mm(acc1, dinv, W1, b1.reshape(1, DH), W2p)
    acc2 = _prop_cp(y2, srcp, dstp, z_fin)
    return _smax(acc2, dinv, b2p)
